# baseline (speedup 1.0000x reference)
import functools

import jax
import jax.numpy as jnp
from jax import lax
from jax.experimental import pallas as pl
from jax.experimental.pallas import tpu as pltpu
from jax.experimental.pallas import tpu_sc as plsc

NDIM = 128
N = 10000
NROWS = 10240
PAD_IDX = 10000
NC = 2
NS = 16
NTILES = NC * NS
E = 320000
EPAD = 327680
CH = 128
CHUNKS = EPAD // CH
IBUF = 8
NCH0 = 40
NCH1 = 120
NSC0 = NCH0 // IBUF
NSC1 = NCH1 // IBUF
STRIPE = NROWS // NS


def _sc_scatter(x, src3, dst3, zacc, ones, with_deg):
    mesh = plsc.VectorSubcoreMesh(core_axis_name="c", subcore_axis_name="s")
    out_type = jax.ShapeDtypeStruct((NC, NROWS, NDIM), jnp.float32)
    if with_deg:
        out_type = (out_type, jax.ShapeDtypeStruct((NC, NROWS, NDIM),
                                                   jnp.float32))

    @functools.partial(
        pl.kernel,
        out_type=out_type,
        mesh=mesh,
        scratch_types=[
            pltpu.VMEM((IBUF, CH), jnp.int32),
            pltpu.VMEM((IBUF, CH), jnp.int32),
            pltpu.VMEM((CH, NDIM), jnp.float32),
            pltpu.VMEM((CH, NDIM), jnp.float32),
            pltpu.VMEM_SHARED((NROWS, NDIM), jnp.float32),
            pltpu.SemaphoreType.DMA,
            pltpu.SemaphoreType.DMA,
            pltpu.SemaphoreType.DMA,
            pltpu.SemaphoreType.DMA,
        ],
    )
    def k(x_hbm, src_hbm, dst_hbm, zacc_hbm, ones_hbm, *refs):
        if with_deg:
            acc_out, deg_out = refs[0], refs[1]
            rest = refs[2:]
        else:
            acc_out = refs[0]
            rest = refs[1:]
        (src_v, dst_v, rows_a, rows_b, acc_sh,
         sem_a, sem_b, sem_c, sem_d) = rest
        cid = lax.axis_index("c")
        sid = lax.axis_index("s")
        cbase = jnp.where(cid == 0, sid * NCH0, NS * NCH0 + sid * NCH1)
        nsc = jnp.where(cid == 0, NSC0, NSC1)
        pltpu.sync_copy(zacc_hbm.at[pl.ds(sid * STRIPE, STRIPE)],
                        acc_sh.at[pl.ds(sid * STRIPE, STRIPE)])

        plsc.subcore_barrier()

        bufs = (rows_a, rows_b)
        gsems = (sem_a, sem_b)
        ssems = (sem_c, sem_d)
        NT = 2 * IBUF

        def gather_ref(t):
            return (src_v if t % 2 == 0 else dst_v).at[t // 2]

        def scatter_ref(t):
            return (dst_v if t % 2 == 0 else src_v).at[t // 2]

        def outer(sc, carry):
            base = cbase + sc * IBUF
            pltpu.sync_copy(src_hbm.at[pl.ds(base, IBUF)], src_v)
            pltpu.sync_copy(dst_hbm.at[pl.ds(base, IBUF)], dst_v)
            cps = {0: pltpu.async_copy(x_hbm.at[gather_ref(0)], bufs[0],
                                       gsems[0])}
            scs = {}
            for t in range(NT):
                if t + 1 < NT:
                    if t - 1 >= 0:
                        scs[t - 1].wait()
                    cps[t + 1] = pltpu.async_copy(
                        x_hbm.at[gather_ref(t + 1)], bufs[(t + 1) % 2],
                        gsems[(t + 1) % 2])
                cps[t].wait()
                scs[t] = pltpu.async_copy(
                    bufs[t % 2], acc_sh.at[scatter_ref(t)], ssems[t % 2],
                    add=True)
            scs[NT - 2].wait()
            scs[NT - 1].wait()
            return carry

        lax.fori_loop(0, nsc, outer, 0)

        plsc.subcore_barrier()

        pltpu.sync_copy(acc_sh.at[pl.ds(sid * STRIPE, STRIPE)],
                        acc_out.at[cid, pl.ds(sid * STRIPE, STRIPE)])

        if with_deg:
            pltpu.sync_copy(zacc_hbm.at[pl.ds(sid * STRIPE, STRIPE)],
                            acc_sh.at[pl.ds(sid * STRIPE, STRIPE)])
            pltpu.sync_copy(ones_hbm, rows_a)
            plsc.subcore_barrier()

            def outer_deg(sc, carry):
                base = cbase + sc * IBUF
                pltpu.sync_copy(src_hbm.at[pl.ds(base, IBUF)], src_v)
                pltpu.sync_copy(dst_hbm.at[pl.ds(base, IBUF)], dst_v)
                ds = [pltpu.async_copy(rows_a, acc_sh.at[scatter_ref(t)],
                                       sem_c, add=True)
                      for t in range(NT)]
                for d in ds:
                    d.wait()
                return carry

            lax.fori_loop(0, nsc, outer_deg, 0)

            plsc.subcore_barrier()
            pltpu.sync_copy(acc_sh.at[pl.ds(sid * STRIPE, STRIPE)],
                            deg_out.at[cid, pl.ds(sid * STRIPE, STRIPE)])

    return k(x, src3, dst3, zacc, ones)


def _tc_mm2(hp, Wn, bn, Ws, bs):
    BM = 1024

    def body(h_ref, wn_ref, bn_ref, ws_ref, bs_ref, x_ref, s_ref):
        hb = h_ref[...]
        x_ref[...] = jnp.dot(hb, wn_ref[...],
                             preferred_element_type=jnp.float32) + bn_ref[...]
        s_ref[...] = jnp.dot(hb, ws_ref[...],
                             preferred_element_type=jnp.float32) + bs_ref[...]

    return pl.pallas_call(
        body,
        grid=(NROWS // BM,),
        in_specs=[
            pl.BlockSpec((BM, NDIM), lambda i: (i, 0)),
            pl.BlockSpec((NDIM, NDIM), lambda i: (0, 0)),
            pl.BlockSpec((1, NDIM), lambda i: (0, 0)),
            pl.BlockSpec((NDIM, NDIM), lambda i: (0, 0)),
            pl.BlockSpec((1, NDIM), lambda i: (0, 0)),
        ],
        out_specs=[
            pl.BlockSpec((BM, NDIM), lambda i: (i, 0)),
            pl.BlockSpec((BM, NDIM), lambda i: (i, 0)),
        ],
        out_shape=[jax.ShapeDtypeStruct((NROWS, NDIM), jnp.float32)] * 2,
    )(hp, Wn, bn.reshape(1, NDIM), Ws, bs.reshape(1, NDIM))


def _tc_update_mm2(s1, a0, a1, dg0, dg1, Wn, bn, Ws, bs):
    BM = 1024

    def body(s_ref, a0_ref, a1_ref, d0_ref, d1_ref, wn_ref, bn_ref,
             ws_ref, bs_ref, x_ref, s2_ref):
        d = d0_ref[:, :1] + d1_ref[:, :1]
        rdeg = 1.0 / jnp.maximum(d, 1.0)
        h1 = jnp.maximum(
            s_ref[...] + (a0_ref[...] + a1_ref[...]) * rdeg, 0.0)
        x_ref[...] = jnp.dot(h1, wn_ref[...],
                             preferred_element_type=jnp.float32) + bn_ref[...]
        s2_ref[...] = jnp.dot(h1, ws_ref[...],
                              preferred_element_type=jnp.float32) + bs_ref[...]

    mspec = pl.BlockSpec((BM, NDIM), lambda i: (i, 0))
    return pl.pallas_call(
        body,
        grid=(NROWS // BM,),
        in_specs=[
            mspec, mspec, mspec, mspec, mspec,
            pl.BlockSpec((NDIM, NDIM), lambda i: (0, 0)),
            pl.BlockSpec((1, NDIM), lambda i: (0, 0)),
            pl.BlockSpec((NDIM, NDIM), lambda i: (0, 0)),
            pl.BlockSpec((1, NDIM), lambda i: (0, 0)),
        ],
        out_specs=[mspec, mspec],
        out_shape=[jax.ShapeDtypeStruct((NROWS, NDIM), jnp.float32)] * 2,
    )(s1, a0, a1, dg0, dg1, Wn, bn.reshape(1, NDIM), Ws, bs.reshape(1, NDIM))


def _tc_finish(s2, a0, a1, dg0, dg1, Wf, bf, Wgt, bgt, Wf_i, bf_i,
               Wgt_i, bgt_i, seg, nseg):
    def body(s_ref, a0_ref, a1_ref, d0_ref, d1_ref, wf_ref, bf_ref,
             wg_ref, bg_ref, wfi_ref, bfi_ref, wgi_ref, bgi_ref,
             hh_ref, hg_ref, hgi_ref):
        d = d0_ref[:, :1] + d1_ref[:, :1]
        rdeg = 1.0 / jnp.maximum(d, 1.0)
        h2 = jnp.maximum(
            s_ref[...] + (a0_ref[...] + a1_ref[...]) * rdeg, 0.0)
        hh_ref[...] = h2[None]
        f = jnp.dot(h2, wf_ref[...],
                    preferred_element_type=jnp.float32) + bf_ref[...]
        g = jax.nn.sigmoid(jnp.dot(h2, wg_ref[...],
                                   preferred_element_type=jnp.float32)
                           + bg_ref[...])
        hg_ref[...] = jnp.sum(f * g, axis=0, keepdims=True)[None]
        fi = jnp.dot(h2, wfi_ref[...],
                     preferred_element_type=jnp.float32) + bfi_ref[...]
        gi = jax.nn.sigmoid(jnp.dot(h2, wgi_ref[...],
                                    preferred_element_type=jnp.float32)
                            + bgi_ref[...])
        hgi_ref[...] = jnp.sum(fi * gi, axis=0, keepdims=True)[None]

    mspec = pl.BlockSpec((seg, NDIM), lambda i: (i, 0))
    wspec = pl.BlockSpec((NDIM, NDIM), lambda i: (0, 0))
    bspec = pl.BlockSpec((1, NDIM), lambda i: (0, 0))
    return pl.pallas_call(
        body,
        grid=(nseg,),
        in_specs=[
            mspec, mspec, mspec, mspec, mspec,
            wspec, bspec, wspec, bspec, wspec, bspec, wspec, bspec,
        ],
        out_specs=[
            pl.BlockSpec((1, seg, NDIM), lambda i: (i, 0, 0)),
            pl.BlockSpec((1, 1, NDIM), lambda i: (i, 0, 0)),
            pl.BlockSpec((1, 1, NDIM), lambda i: (i, 0, 0)),
        ],
        out_shape=[
            jax.ShapeDtypeStruct((nseg, seg, NDIM), jnp.float32),
            jax.ShapeDtypeStruct((nseg, 1, NDIM), jnp.float32),
            jax.ShapeDtypeStruct((nseg, 1, NDIM), jnp.float32),
        ],
    )(s2, a0, a1, dg0, dg1, Wf, bf.reshape(1, NDIM), Wgt, bgt, Wf_i,
      bf_i.reshape(1, NDIM), Wgt_i, bgt_i)


def kernel(h, edge_index, W_self1, b_self1, W_nbr1, b_nbr1,
           W_self2, b_self2, W_nbr2, b_nbr2,
           Wf, bf, Wg, bg, Wf_i, bf_i, Wg_i, bg_i):
    nseg, seg, ndim = h.shape
    hf = h.reshape(-1, ndim)
    hp = jnp.concatenate(
        [hf, jnp.zeros((NROWS - N, ndim), jnp.float32)], axis=0)

    pad_col = jnp.full((EPAD - E,), PAD_IDX, jnp.int32)
    src3 = jnp.concatenate([edge_index[0], pad_col]).reshape(CHUNKS, CH)
    dst3 = jnp.concatenate([edge_index[1], pad_col]).reshape(CHUNKS, CH)

    zacc = jnp.zeros((NROWS, NDIM), jnp.float32)
    ones = jnp.ones((CH, NDIM), jnp.float32)

    x1, s1 = _tc_mm2(hp, W_nbr1, b_nbr1, W_self1, b_self1)
    acc1, deg = _sc_scatter(x1, src3, dst3, zacc, ones, with_deg=True)

    x2, s2 = _tc_update_mm2(s1, acc1[0], acc1[1], deg[0], deg[1],
                            W_nbr2, b_nbr2, W_self2, b_self2)
    acc2 = _sc_scatter(x2, src3, dst3, zacc, ones, with_deg=False)

    Wgt = jnp.tile(Wg, (1, NDIM))
    bgt = jnp.broadcast_to(bg, (1, NDIM))
    Wgt_i = jnp.tile(Wg_i, (1, NDIM))
    bgt_i = jnp.broadcast_to(bg_i, (1, NDIM))

    hh, h_G, h_G_init = _tc_finish(
        s2[:N], acc2[0, :N], acc2[1, :N], deg[0, :N], deg[1, :N],
        Wf, bf, Wgt, bgt, Wf_i, bf_i, Wgt_i, bgt_i, seg, nseg)
    return hh, h_G[:, 0, :], h_G_init[:, 0, :]


# speedup vs baseline: 1.0806x; 1.0806x over previous
import functools

import jax
import jax.numpy as jnp
from jax import lax
from jax.experimental import pallas as pl
from jax.experimental.pallas import tpu as pltpu
from jax.experimental.pallas import tpu_sc as plsc

NDIM = 128
N = 10000
NROWS = 10240
PAD_IDX = 10000
NC = 2
NS = 16
NTILES = NC * NS
E = 320000
EPAD = 327680
CH = 128
CHUNKS = EPAD // CH
IBUF = 8
NCH0 = 120
NCH1 = 40
NSC0 = NCH0 // IBUF
NSC1 = NCH1 // IBUF
STRIPE = NROWS // NS


def _sc_scatter(x, src3, dst3, with_deg):
    mesh = plsc.VectorSubcoreMesh(core_axis_name="c", subcore_axis_name="s")
    out_type = jax.ShapeDtypeStruct((NC, NROWS, NDIM), jnp.float32)
    if with_deg:
        out_type = (out_type, jax.ShapeDtypeStruct((NC, NROWS, NDIM),
                                                   jnp.float32))

    @functools.partial(
        pl.kernel,
        out_type=out_type,
        mesh=mesh,
        scratch_types=[
            pltpu.VMEM((IBUF, CH), jnp.int32),
            pltpu.VMEM((IBUF, CH), jnp.int32),
            pltpu.VMEM((CH, NDIM), jnp.float32),
            pltpu.VMEM((CH, NDIM), jnp.float32),
            pltpu.VMEM_SHARED((NROWS, NDIM), jnp.float32),
            pltpu.SemaphoreType.DMA,
            pltpu.SemaphoreType.DMA,
            pltpu.SemaphoreType.DMA,
            pltpu.SemaphoreType.DMA,
        ],
    )
    def k(x_hbm, src_hbm, dst_hbm, *refs):
        if with_deg:
            acc_out, deg_out = refs[0], refs[1]
            rest = refs[2:]
        else:
            acc_out = refs[0]
            rest = refs[1:]
        (src_v, dst_v, rows_a, rows_b, acc_sh,
         sem_a, sem_b, sem_c, sem_d) = rest
        cid = lax.axis_index("c")
        sid = lax.axis_index("s")
        cbase = jnp.where(cid == 0, sid * NCH0, NS * NCH0 + sid * NCH1)
        nsc = jnp.where(cid == 0, NSC0, NSC1)

        def _fill(buf, value):
            v16 = jnp.full((16,), value, jnp.float32)

            def frow(r, c):
                for j in range(NDIM // 16):
                    buf[r, pl.ds(j * 16, 16)] = v16
                return c

            lax.fori_loop(0, CH, frow, 0)

        def _zero_stripe(buf):
            for kk in range(STRIPE // CH):
                pltpu.sync_copy(
                    buf, acc_sh.at[pl.ds(sid * STRIPE + kk * CH, CH)])

        _fill(rows_a, 0.0)
        _zero_stripe(rows_a)

        plsc.subcore_barrier()

        bufs = (rows_a, rows_b)
        gsems = (sem_a, sem_b)
        ssems = (sem_c, sem_d)
        NT = 2 * IBUF

        def gather_ref(t):
            return (src_v if t % 2 == 0 else dst_v).at[t // 2]

        def scatter_ref(t):
            return (dst_v if t % 2 == 0 else src_v).at[t // 2]

        def outer(sc, carry):
            base = cbase + sc * IBUF
            pltpu.sync_copy(src_hbm.at[pl.ds(base, IBUF)], src_v)
            pltpu.sync_copy(dst_hbm.at[pl.ds(base, IBUF)], dst_v)
            cps = {0: pltpu.async_copy(x_hbm.at[gather_ref(0)], bufs[0],
                                       gsems[0])}
            scs = {}
            for t in range(NT):
                if t + 1 < NT:
                    if t - 1 >= 0:
                        scs[t - 1].wait()
                    cps[t + 1] = pltpu.async_copy(
                        x_hbm.at[gather_ref(t + 1)], bufs[(t + 1) % 2],
                        gsems[(t + 1) % 2])
                cps[t].wait()
                scs[t] = pltpu.async_copy(
                    bufs[t % 2], acc_sh.at[scatter_ref(t)], ssems[t % 2],
                    add=True)
            scs[NT - 2].wait()
            scs[NT - 1].wait()
            return carry

        lax.fori_loop(0, nsc, outer, 0)

        plsc.subcore_barrier()

        pltpu.sync_copy(acc_sh.at[pl.ds(sid * STRIPE, STRIPE)],
                        acc_out.at[cid, pl.ds(sid * STRIPE, STRIPE)])

        if with_deg:
            _fill(rows_b, 0.0)
            _zero_stripe(rows_b)
            _fill(rows_a, 1.0)
            plsc.subcore_barrier()

            def outer_deg(sc, carry):
                base = cbase + sc * IBUF
                pltpu.sync_copy(src_hbm.at[pl.ds(base, IBUF)], src_v)
                pltpu.sync_copy(dst_hbm.at[pl.ds(base, IBUF)], dst_v)
                ds = [pltpu.async_copy(rows_a, acc_sh.at[scatter_ref(t)],
                                       sem_c, add=True)
                      for t in range(NT)]
                for d in ds:
                    d.wait()
                return carry

            lax.fori_loop(0, nsc, outer_deg, 0)

            plsc.subcore_barrier()
            pltpu.sync_copy(acc_sh.at[pl.ds(sid * STRIPE, STRIPE)],
                            deg_out.at[cid, pl.ds(sid * STRIPE, STRIPE)])

    return k(x, src3, dst3)


def _tc_mm2(hp, Wn, bn, Ws, bs):
    BM = 1024

    def body(h_ref, wn_ref, bn_ref, ws_ref, bs_ref, x_ref, s_ref):
        hb = h_ref[...]
        x_ref[...] = jnp.dot(hb, wn_ref[...],
                             preferred_element_type=jnp.float32) + bn_ref[...]
        s_ref[...] = jnp.dot(hb, ws_ref[...],
                             preferred_element_type=jnp.float32) + bs_ref[...]

    return pl.pallas_call(
        body,
        grid=(NROWS // BM,),
        in_specs=[
            pl.BlockSpec((BM, NDIM), lambda i: (i, 0)),
            pl.BlockSpec((NDIM, NDIM), lambda i: (0, 0)),
            pl.BlockSpec((1, NDIM), lambda i: (0, 0)),
            pl.BlockSpec((NDIM, NDIM), lambda i: (0, 0)),
            pl.BlockSpec((1, NDIM), lambda i: (0, 0)),
        ],
        out_specs=[
            pl.BlockSpec((BM, NDIM), lambda i: (i, 0)),
            pl.BlockSpec((BM, NDIM), lambda i: (i, 0)),
        ],
        out_shape=[jax.ShapeDtypeStruct((NROWS, NDIM), jnp.float32)] * 2,
    )(hp, Wn, bn.reshape(1, NDIM), Ws, bs.reshape(1, NDIM))


def _tc_update_mm2(s1, a0, a1, dg0, dg1, Wn, bn, Ws, bs):
    BM = 1024

    def body(s_ref, a0_ref, a1_ref, d0_ref, d1_ref, wn_ref, bn_ref,
             ws_ref, bs_ref, x_ref, s2_ref):
        d = d0_ref[:, :1] + d1_ref[:, :1]
        rdeg = 1.0 / jnp.maximum(d, 1.0)
        h1 = jnp.maximum(
            s_ref[...] + (a0_ref[...] + a1_ref[...]) * rdeg, 0.0)
        x_ref[...] = jnp.dot(h1, wn_ref[...],
                             preferred_element_type=jnp.float32) + bn_ref[...]
        s2_ref[...] = jnp.dot(h1, ws_ref[...],
                              preferred_element_type=jnp.float32) + bs_ref[...]

    mspec = pl.BlockSpec((BM, NDIM), lambda i: (i, 0))
    return pl.pallas_call(
        body,
        grid=(NROWS // BM,),
        in_specs=[
            mspec, mspec, mspec, mspec, mspec,
            pl.BlockSpec((NDIM, NDIM), lambda i: (0, 0)),
            pl.BlockSpec((1, NDIM), lambda i: (0, 0)),
            pl.BlockSpec((NDIM, NDIM), lambda i: (0, 0)),
            pl.BlockSpec((1, NDIM), lambda i: (0, 0)),
        ],
        out_specs=[mspec, mspec],
        out_shape=[jax.ShapeDtypeStruct((NROWS, NDIM), jnp.float32)] * 2,
    )(s1, a0, a1, dg0, dg1, Wn, bn.reshape(1, NDIM), Ws, bs.reshape(1, NDIM))


def _tc_finish(s2, a0, a1, dg0, dg1, Wf, bf, Wgt, bgt, Wf_i, bf_i,
               Wgt_i, bgt_i, seg, nseg):
    def body(s_ref, a0_ref, a1_ref, d0_ref, d1_ref, wf_ref, bf_ref,
             wg_ref, bg_ref, wfi_ref, bfi_ref, wgi_ref, bgi_ref,
             hh_ref, hg_ref, hgi_ref):
        d = d0_ref[:, :1] + d1_ref[:, :1]
        rdeg = 1.0 / jnp.maximum(d, 1.0)
        h2 = jnp.maximum(
            s_ref[...] + (a0_ref[...] + a1_ref[...]) * rdeg, 0.0)
        hh_ref[...] = h2[None]
        f = jnp.dot(h2, wf_ref[...],
                    preferred_element_type=jnp.float32) + bf_ref[...]
        g = jax.nn.sigmoid(jnp.dot(h2, wg_ref[...],
                                   preferred_element_type=jnp.float32)
                           + bg_ref[...])
        hg_ref[...] = jnp.sum(f * g, axis=0, keepdims=True)[None]
        fi = jnp.dot(h2, wfi_ref[...],
                     preferred_element_type=jnp.float32) + bfi_ref[...]
        gi = jax.nn.sigmoid(jnp.dot(h2, wgi_ref[...],
                                    preferred_element_type=jnp.float32)
                            + bgi_ref[...])
        hgi_ref[...] = jnp.sum(fi * gi, axis=0, keepdims=True)[None]

    mspec = pl.BlockSpec((seg, NDIM), lambda i: (i, 0))
    wspec = pl.BlockSpec((NDIM, NDIM), lambda i: (0, 0))
    bspec = pl.BlockSpec((1, NDIM), lambda i: (0, 0))
    return pl.pallas_call(
        body,
        grid=(nseg,),
        in_specs=[
            mspec, mspec, mspec, mspec, mspec,
            wspec, bspec, wspec, bspec, wspec, bspec, wspec, bspec,
        ],
        out_specs=[
            pl.BlockSpec((1, seg, NDIM), lambda i: (i, 0, 0)),
            pl.BlockSpec((1, 1, NDIM), lambda i: (i, 0, 0)),
            pl.BlockSpec((1, 1, NDIM), lambda i: (i, 0, 0)),
        ],
        out_shape=[
            jax.ShapeDtypeStruct((nseg, seg, NDIM), jnp.float32),
            jax.ShapeDtypeStruct((nseg, 1, NDIM), jnp.float32),
            jax.ShapeDtypeStruct((nseg, 1, NDIM), jnp.float32),
        ],
    )(s2, a0, a1, dg0, dg1, Wf, bf.reshape(1, NDIM), Wgt, bgt, Wf_i,
      bf_i.reshape(1, NDIM), Wgt_i, bgt_i)


def kernel(h, edge_index, W_self1, b_self1, W_nbr1, b_nbr1,
           W_self2, b_self2, W_nbr2, b_nbr2,
           Wf, bf, Wg, bg, Wf_i, bf_i, Wg_i, bg_i):
    nseg, seg, ndim = h.shape
    hf = h.reshape(-1, ndim)
    hp = jnp.concatenate(
        [hf, jnp.zeros((NROWS - N, ndim), jnp.float32)], axis=0)

    pad_col = jnp.full((EPAD - E,), PAD_IDX, jnp.int32)
    src3 = jnp.concatenate([edge_index[0], pad_col]).reshape(CHUNKS, CH)
    dst3 = jnp.concatenate([edge_index[1], pad_col]).reshape(CHUNKS, CH)

    x1, s1 = _tc_mm2(hp, W_nbr1, b_nbr1, W_self1, b_self1)
    acc1, deg = _sc_scatter(x1, src3, dst3, with_deg=True)

    x2, s2 = _tc_update_mm2(s1, acc1[0], acc1[1], deg[0], deg[1],
                            W_nbr2, b_nbr2, W_self2, b_self2)
    acc2 = _sc_scatter(x2, src3, dst3, with_deg=False)

    Wgt = jnp.tile(Wg, (1, NDIM))
    bgt = jnp.broadcast_to(bg, (1, NDIM))
    Wgt_i = jnp.tile(Wg_i, (1, NDIM))
    bgt_i = jnp.broadcast_to(bg_i, (1, NDIM))

    hh, h_G, h_G_init = _tc_finish(
        s2[:N], acc2[0, :N], acc2[1, :N], deg[0, :N], deg[1, :N],
        Wf, bf, Wgt, bgt, Wf_i, bf_i, Wgt_i, bgt_i, seg, nseg)
    return hh, h_G[:, 0, :], h_G_init[:, 0, :]


# speedup vs baseline: 1.2545x; 1.1609x over previous
import functools

import jax
import jax.numpy as jnp
from jax import lax
from jax.experimental import pallas as pl
from jax.experimental.pallas import tpu as pltpu
from jax.experimental.pallas import tpu_sc as plsc

NDIM = 128
N = 10000
NROWS = 10240
PAD_IDX = 10000
NC = 2
NS = 16
NTILES = NC * NS
E = 320000
EPAD = 327680
CH = 128
CHUNKS = EPAD // CH
IBUF = 8
NCH0 = 120
NCH1 = 40
NSC0 = NCH0 // IBUF
NSC1 = NCH1 // IBUF
STRIPE = NROWS // NS


def _sc_scatter(x, src3, dst3, zrows, orows, with_deg):
    mesh = plsc.VectorSubcoreMesh(core_axis_name="c", subcore_axis_name="s")
    out_type = jax.ShapeDtypeStruct((NC, NROWS, NDIM), jnp.float32)
    if with_deg:
        out_type = (out_type, jax.ShapeDtypeStruct((NC, NROWS, NDIM),
                                                   jnp.float32))

    @functools.partial(
        pl.kernel,
        out_type=out_type,
        mesh=mesh,
        scratch_types=[
            pltpu.VMEM((IBUF, CH), jnp.int32),
            pltpu.VMEM((IBUF, CH), jnp.int32),
            pltpu.VMEM((CH, NDIM), jnp.float32),
            pltpu.VMEM((CH, NDIM), jnp.float32),
            pltpu.VMEM((64, NDIM), jnp.float32),
            pltpu.VMEM_SHARED((NROWS, NDIM), jnp.float32),
            pltpu.SemaphoreType.DMA,
            pltpu.SemaphoreType.DMA,
            pltpu.SemaphoreType.DMA,
            pltpu.SemaphoreType.DMA,
        ],
    )
    def k(x_hbm, src_hbm, dst_hbm, zrows_hbm, orows_hbm, *refs):
        if with_deg:
            acc_out, deg_out = refs[0], refs[1]
            rest = refs[2:]
        else:
            acc_out = refs[0]
            rest = refs[1:]
        (src_v, dst_v, rows_a, rows_b, zbuf, acc_sh,
         sem_a, sem_b, sem_c, sem_d) = rest
        cid = lax.axis_index("c")
        sid = lax.axis_index("s")
        cbase = jnp.where(cid == 0, sid * NCH0, NS * NCH0 + sid * NCH1)
        nsc = jnp.where(cid == 0, NSC0, NSC1)

        def _zero_stripe():
            for kk in range(STRIPE // 64):
                pltpu.sync_copy(
                    zbuf, acc_sh.at[pl.ds(sid * STRIPE + kk * 64, 64)])

        pltpu.sync_copy(zrows_hbm, zbuf)
        _zero_stripe()

        plsc.subcore_barrier()

        bufs = (rows_a, rows_b)
        gsems = (sem_a, sem_b)
        ssems = (sem_c, sem_d)
        NT = 2 * IBUF

        def gather_ref(t):
            return (src_v if t % 2 == 0 else dst_v).at[t // 2]

        def scatter_ref(t):
            return (dst_v if t % 2 == 0 else src_v).at[t // 2]

        def outer(sc, carry):
            base = cbase + sc * IBUF
            pltpu.sync_copy(src_hbm.at[pl.ds(base, IBUF)], src_v)
            pltpu.sync_copy(dst_hbm.at[pl.ds(base, IBUF)], dst_v)
            cps = {0: pltpu.async_copy(x_hbm.at[gather_ref(0)], bufs[0],
                                       gsems[0])}
            scs = {}
            for t in range(NT):
                if t + 1 < NT:
                    if t - 1 >= 0:
                        scs[t - 1].wait()
                    cps[t + 1] = pltpu.async_copy(
                        x_hbm.at[gather_ref(t + 1)], bufs[(t + 1) % 2],
                        gsems[(t + 1) % 2])
                cps[t].wait()
                scs[t] = pltpu.async_copy(
                    bufs[t % 2], acc_sh.at[scatter_ref(t)], ssems[t % 2],
                    add=True)
            scs[NT - 2].wait()
            scs[NT - 1].wait()
            return carry

        lax.fori_loop(0, nsc, outer, 0)

        plsc.subcore_barrier()

        pltpu.sync_copy(acc_sh.at[pl.ds(sid * STRIPE, STRIPE)],
                        acc_out.at[cid, pl.ds(sid * STRIPE, STRIPE)])

        if with_deg:
            _zero_stripe()
            pltpu.sync_copy(orows_hbm, rows_a)
            plsc.subcore_barrier()

            def outer_deg(sc, carry):
                base = cbase + sc * IBUF
                pltpu.sync_copy(src_hbm.at[pl.ds(base, IBUF)], src_v)
                pltpu.sync_copy(dst_hbm.at[pl.ds(base, IBUF)], dst_v)
                ds = [pltpu.async_copy(rows_a, acc_sh.at[scatter_ref(t)],
                                       sem_c, add=True)
                      for t in range(NT)]
                for d in ds:
                    d.wait()
                return carry

            lax.fori_loop(0, nsc, outer_deg, 0)

            plsc.subcore_barrier()
            pltpu.sync_copy(acc_sh.at[pl.ds(sid * STRIPE, STRIPE)],
                            deg_out.at[cid, pl.ds(sid * STRIPE, STRIPE)])

    return k(x, src3, dst3, zrows, orows)


def _tc_mm2(hp, Wn, bn, Ws, bs):
    BM = 1024

    def body(h_ref, wn_ref, bn_ref, ws_ref, bs_ref, x_ref, s_ref):
        hb = h_ref[...]
        x_ref[...] = jnp.dot(hb, wn_ref[...],
                             preferred_element_type=jnp.float32) + bn_ref[...]
        s_ref[...] = jnp.dot(hb, ws_ref[...],
                             preferred_element_type=jnp.float32) + bs_ref[...]

    return pl.pallas_call(
        body,
        grid=(NROWS // BM,),
        in_specs=[
            pl.BlockSpec((BM, NDIM), lambda i: (i, 0)),
            pl.BlockSpec((NDIM, NDIM), lambda i: (0, 0)),
            pl.BlockSpec((1, NDIM), lambda i: (0, 0)),
            pl.BlockSpec((NDIM, NDIM), lambda i: (0, 0)),
            pl.BlockSpec((1, NDIM), lambda i: (0, 0)),
        ],
        out_specs=[
            pl.BlockSpec((BM, NDIM), lambda i: (i, 0)),
            pl.BlockSpec((BM, NDIM), lambda i: (i, 0)),
        ],
        out_shape=[jax.ShapeDtypeStruct((NROWS, NDIM), jnp.float32)] * 2,
    )(hp, Wn, bn.reshape(1, NDIM), Ws, bs.reshape(1, NDIM))


def _tc_update_mm2(s1, a0, a1, dg0, dg1, Wn, bn, Ws, bs):
    BM = 1024

    def body(s_ref, a0_ref, a1_ref, d0_ref, d1_ref, wn_ref, bn_ref,
             ws_ref, bs_ref, x_ref, s2_ref):
        d = d0_ref[:, :1] + d1_ref[:, :1]
        rdeg = 1.0 / jnp.maximum(d, 1.0)
        h1 = jnp.maximum(
            s_ref[...] + (a0_ref[...] + a1_ref[...]) * rdeg, 0.0)
        x_ref[...] = jnp.dot(h1, wn_ref[...],
                             preferred_element_type=jnp.float32) + bn_ref[...]
        s2_ref[...] = jnp.dot(h1, ws_ref[...],
                              preferred_element_type=jnp.float32) + bs_ref[...]

    mspec = pl.BlockSpec((BM, NDIM), lambda i: (i, 0))
    return pl.pallas_call(
        body,
        grid=(NROWS // BM,),
        in_specs=[
            mspec, mspec, mspec, mspec, mspec,
            pl.BlockSpec((NDIM, NDIM), lambda i: (0, 0)),
            pl.BlockSpec((1, NDIM), lambda i: (0, 0)),
            pl.BlockSpec((NDIM, NDIM), lambda i: (0, 0)),
            pl.BlockSpec((1, NDIM), lambda i: (0, 0)),
        ],
        out_specs=[mspec, mspec],
        out_shape=[jax.ShapeDtypeStruct((NROWS, NDIM), jnp.float32)] * 2,
    )(s1, a0, a1, dg0, dg1, Wn, bn.reshape(1, NDIM), Ws, bs.reshape(1, NDIM))


def _tc_finish(s2, a0, a1, dg0, dg1, Wf, bf, Wgt, bgt, Wf_i, bf_i,
               Wgt_i, bgt_i, seg, nseg):
    def body(s_ref, a0_ref, a1_ref, d0_ref, d1_ref, wf_ref, bf_ref,
             wg_ref, bg_ref, wfi_ref, bfi_ref, wgi_ref, bgi_ref,
             hh_ref, hg_ref, hgi_ref):
        d = d0_ref[:, :1] + d1_ref[:, :1]
        rdeg = 1.0 / jnp.maximum(d, 1.0)
        h2 = jnp.maximum(
            s_ref[...] + (a0_ref[...] + a1_ref[...]) * rdeg, 0.0)
        hh_ref[...] = h2[None]
        f = jnp.dot(h2, wf_ref[...],
                    preferred_element_type=jnp.float32) + bf_ref[...]
        g = jax.nn.sigmoid(jnp.dot(h2, wg_ref[...],
                                   preferred_element_type=jnp.float32)
                           + bg_ref[...])
        hg_ref[...] = jnp.sum(f * g, axis=0, keepdims=True)[None]
        fi = jnp.dot(h2, wfi_ref[...],
                     preferred_element_type=jnp.float32) + bfi_ref[...]
        gi = jax.nn.sigmoid(jnp.dot(h2, wgi_ref[...],
                                    preferred_element_type=jnp.float32)
                            + bgi_ref[...])
        hgi_ref[...] = jnp.sum(fi * gi, axis=0, keepdims=True)[None]

    mspec = pl.BlockSpec((seg, NDIM), lambda i: (i, 0))
    wspec = pl.BlockSpec((NDIM, NDIM), lambda i: (0, 0))
    bspec = pl.BlockSpec((1, NDIM), lambda i: (0, 0))
    return pl.pallas_call(
        body,
        grid=(nseg,),
        in_specs=[
            mspec, mspec, mspec, mspec, mspec,
            wspec, bspec, wspec, bspec, wspec, bspec, wspec, bspec,
        ],
        out_specs=[
            pl.BlockSpec((1, seg, NDIM), lambda i: (i, 0, 0)),
            pl.BlockSpec((1, 1, NDIM), lambda i: (i, 0, 0)),
            pl.BlockSpec((1, 1, NDIM), lambda i: (i, 0, 0)),
        ],
        out_shape=[
            jax.ShapeDtypeStruct((nseg, seg, NDIM), jnp.float32),
            jax.ShapeDtypeStruct((nseg, 1, NDIM), jnp.float32),
            jax.ShapeDtypeStruct((nseg, 1, NDIM), jnp.float32),
        ],
    )(s2, a0, a1, dg0, dg1, Wf, bf.reshape(1, NDIM), Wgt, bgt, Wf_i,
      bf_i.reshape(1, NDIM), Wgt_i, bgt_i)


def kernel(h, edge_index, W_self1, b_self1, W_nbr1, b_nbr1,
           W_self2, b_self2, W_nbr2, b_nbr2,
           Wf, bf, Wg, bg, Wf_i, bf_i, Wg_i, bg_i):
    nseg, seg, ndim = h.shape
    hf = h.reshape(-1, ndim)
    hp = jnp.concatenate(
        [hf, jnp.zeros((NROWS - N, ndim), jnp.float32)], axis=0)

    pad_col = jnp.full((EPAD - E,), PAD_IDX, jnp.int32)
    src3 = jnp.concatenate([edge_index[0], pad_col]).reshape(CHUNKS, CH)
    dst3 = jnp.concatenate([edge_index[1], pad_col]).reshape(CHUNKS, CH)

    zrows = jnp.zeros((64, NDIM), jnp.float32)
    orows = jnp.ones((CH, NDIM), jnp.float32)

    x1, s1 = _tc_mm2(hp, W_nbr1, b_nbr1, W_self1, b_self1)
    acc1, deg = _sc_scatter(x1, src3, dst3, zrows, orows, with_deg=True)

    x2, s2 = _tc_update_mm2(s1, acc1[0], acc1[1], deg[0], deg[1],
                            W_nbr2, b_nbr2, W_self2, b_self2)
    acc2 = _sc_scatter(x2, src3, dst3, zrows, orows, with_deg=False)

    Wgt = jnp.tile(Wg, (1, NDIM))
    bgt = jnp.broadcast_to(bg, (1, NDIM))
    Wgt_i = jnp.tile(Wg_i, (1, NDIM))
    bgt_i = jnp.broadcast_to(bg_i, (1, NDIM))

    hh, h_G, h_G_init = _tc_finish(
        s2[:N], acc2[0, :N], acc2[1, :N], deg[0, :N], deg[1, :N],
        Wf, bf, Wgt, bgt, Wf_i, bf_i, Wgt_i, bgt_i, seg, nseg)
    return hh, h_G[:, 0, :], h_G_init[:, 0, :]
